# Initial kernel scaffold; baseline (speedup 1.0000x reference)
import jax
import jax.numpy as jnp
from jax.experimental import pallas as pl


def kernel(x, edge_index, W_in, b_in, Wmsg, Wz, Uz, bz, Wr, Ur, br, Wh, Uh, bh, W_out, b_out, Pw1, Pb1, Pw2, Pb2, Rw, Rb):
    raise NotImplementedError("write your pallas kernel here")


# speedup vs baseline: 2.4749x; 2.4749x over previous
import functools

import jax
import jax.numpy as jnp
from jax import lax
from jax.experimental import pallas as pl
from jax.experimental.pallas import tpu as pltpu
from jax.experimental.pallas import tpu_sc as plsc

N = 10000
E = 160000
D_IN = 256
H = 256
HH = 128
D_EMB = 64
NLAYERS = 3
NSTEPS = 5
PH = 256
PO = 64

NC = 2
NS = 16
CHUNK = 128
NCHUNK = 80
EPT = NCHUNK * CHUNK
EPAD = EPT * NS
ZROWS = 640
NPAD = NS * ZROWS
DUMP = N
OPT = 624

BN = 1000
G = N // BN


def _sc_body(mw_hbm, src_hbm, dst_hbm, out_hbm, srcv, dstv, buf, acc, sem):
    c = lax.axis_index("c")
    s = lax.axis_index("s")
    pltpu.sync_copy(src_hbm.at[pl.ds((c * NS + s) * NCHUNK, NCHUNK)], srcv)
    pltpu.sync_copy(dst_hbm.at[pl.ds(s * NCHUNK, NCHUNK)], dstv)
    zero16 = jnp.zeros((16,), jnp.float32)

    def zrow(i, carry):
        for j in range(HH // 16):
            buf[i, pl.ds(j * 16, 16)] = zero16
        return carry

    lax.fori_loop(0, CHUNK, zrow, 0)
    for b in range(ZROWS // CHUNK):
        pltpu.sync_copy(buf, acc.at[pl.ds(s * ZROWS + b * CHUNK, CHUNK)])
    plsc.subcore_barrier()

    def step(j, carry):
        pltpu.async_copy(mw_hbm.at[srcv.at[j]], buf, sem).wait()
        pltpu.sync_copy(buf, acc.at[dstv.at[j]], add=True)
        return carry

    lax.fori_loop(0, NCHUNK, step, 0)
    plsc.subcore_barrier()

    @pl.when(s < NS - 1)
    def _():
        pltpu.sync_copy(acc.at[pl.ds(s * OPT, OPT)],
                        out_hbm.at[pl.ds(c * N + s * OPT, OPT)])

    @pl.when(s == NS - 1)
    def _():
        pltpu.sync_copy(acc.at[pl.ds((NS - 1) * OPT, N - (NS - 1) * OPT)],
                        out_hbm.at[pl.ds(c * N + (NS - 1) * OPT,
                                         N - (NS - 1) * OPT)])


@functools.lru_cache(maxsize=1)
def _sc_segment_sum_kernel():
    mesh = plsc.VectorSubcoreMesh(
        core_axis_name="c", subcore_axis_name="s",
        num_cores=NC, num_subcores=NS)
    return pl.kernel(
        _sc_body,
        out_type=jax.ShapeDtypeStruct((NC * N, HH), jnp.float32),
        mesh=mesh,
        scratch_types=[
            pltpu.VMEM((NCHUNK, CHUNK), jnp.int32),
            pltpu.VMEM((NCHUNK, CHUNK), jnp.int32),
            pltpu.VMEM((CHUNK, HH), jnp.float32),
            pltpu.VMEM_SHARED((NPAD, HH), jnp.float32),
            pltpu.SemaphoreType.DMA,
        ],
    )


def _sc_segment_sum(mw2, src2, dst2):
    return _sc_segment_sum_kernel()(mw2, src2, dst2)


def _full(shape):
    return pl.BlockSpec(shape, lambda i: tuple(0 for _ in shape))


def _rows(shape):
    return pl.BlockSpec(shape, lambda i: (i,) + tuple(0 for _ in shape[1:]))


def _tc_input_proj(x, w_in, b_in, wm0):
    def body(x_ref, w_ref, b_ref, wm_ref, h_ref, mw_ref):
        h = jnp.maximum(jnp.dot(x_ref[...], w_ref[...]) + b_ref[...], 0.0)
        h_ref[...] = h
        mw_ref[0] = jnp.dot(h, wm_ref[:, :HH])
        mw_ref[1] = jnp.dot(h, wm_ref[:, HH:])

    return pl.pallas_call(
        body,
        grid=(G,),
        in_specs=[_rows((BN, D_IN)), _full((D_IN, H)), _full((1, H)),
                  _full((H, H))],
        out_specs=[_rows((BN, H)),
                   pl.BlockSpec((NC, BN, HH), lambda i: (0, i, 0))],
        out_shape=[jax.ShapeDtypeStruct((N, H), jnp.float32),
                   jax.ShapeDtypeStruct((NC, N, HH), jnp.float32)],
    )(x, w_in, b_in, wm0)


def _tc_gru(h, msg2, wz, uz, bz, wr, ur, br, wh, uh, bh, wm_next, h_res):
    with_mw = wm_next is not None
    with_res = h_res is not None

    def body(*refs):
        it = iter(refs)
        h_ref = next(it); m0_ref = next(it); m1_ref = next(it)
        wz_ref = next(it); uz_ref = next(it); bz_ref = next(it)
        wr_ref = next(it); ur_ref = next(it); br_ref = next(it)
        wh_ref = next(it); uh_ref = next(it); bh_ref = next(it)
        wm_ref = next(it) if with_mw else None
        hr_ref = next(it) if with_res else None
        out_h = next(it)
        out_mw = next(it) if with_mw else None

        h = h_ref[...]
        m0 = m0_ref[...]
        m1 = m1_ref[...]
        z = jax.nn.sigmoid(jnp.dot(m0, wz_ref[:HH, :]) + jnp.dot(m1, wz_ref[HH:, :])
                           + jnp.dot(h, uz_ref[...]) + bz_ref[...])
        r = jax.nn.sigmoid(jnp.dot(m0, wr_ref[:HH, :]) + jnp.dot(m1, wr_ref[HH:, :])
                           + jnp.dot(h, ur_ref[...]) + br_ref[...])
        ht = jnp.tanh(jnp.dot(m0, wh_ref[:HH, :]) + jnp.dot(m1, wh_ref[HH:, :])
                      + jnp.dot(r * h, uh_ref[...]) + bh_ref[...])
        hn = (1.0 - z) * h + z * ht
        if with_res:
            hn = hn + hr_ref[...]
        out_h[...] = hn
        if with_mw:
            out_mw[0] = jnp.dot(hn, wm_ref[:, :HH])
            out_mw[1] = jnp.dot(hn, wm_ref[:, HH:])

    in_specs = [_rows((BN, H)),
                pl.BlockSpec((BN, HH), lambda i: (i, 0)),
                pl.BlockSpec((BN, HH), lambda i: (G + i, 0)),
                _full((H, H)), _full((H, H)), _full((1, H)),
                _full((H, H)), _full((H, H)), _full((1, H)),
                _full((H, H)), _full((H, H)), _full((1, H))]
    args = [h, msg2, msg2, wz, uz, bz, wr, ur, br, wh, uh, bh]
    if with_mw:
        in_specs.append(_full((H, H)))
        args.append(wm_next)
    if with_res:
        in_specs.append(_rows((BN, H)))
        args.append(h_res)
    out_specs = [_rows((BN, H))]
    out_shape = [jax.ShapeDtypeStruct((N, H), jnp.float32)]
    if with_mw:
        out_specs.append(pl.BlockSpec((NC, BN, HH), lambda i: (0, i, 0)))
        out_shape.append(jax.ShapeDtypeStruct((NC, N, HH), jnp.float32))
    res = pl.pallas_call(
        body, grid=(G,), in_specs=in_specs, out_specs=out_specs,
        out_shape=out_shape,
    )(*args)
    return (res[0], res[1]) if with_mw else (res[0], None)


def _tc_tail(h, w_out, b_out, pw1, pb1, pw2, pb2, rw, rb):
    def body(h_ref, wo_ref, bo_ref, p1_ref, pb1_ref, p2_ref, pb2_ref,
             rw_ref, rb_ref, ne_ref, pe_ref, rk_ref):
        ne = jnp.dot(h_ref[...], wo_ref[...]) + bo_ref[...]
        ne_ref[...] = ne
        hid = jnp.maximum(jnp.dot(ne, p1_ref[...]) + pb1_ref[...], 0.0)
        pe_ref[...] = jnp.dot(hid, p2_ref[...]) + pb2_ref[...]
        rk_ref[...] = jnp.dot(ne, rw_ref[...]) + rb_ref[...]

    return pl.pallas_call(
        body,
        grid=(G,),
        in_specs=[_rows((BN, H)), _full((H, D_EMB)), _full((1, D_EMB)),
                  _full((D_EMB, PH)), _full((1, PH)),
                  _full((PH, PO)), _full((1, PO)),
                  _full((D_EMB, 1)), _full((1, 1))],
        out_specs=[_rows((BN, D_EMB)), _rows((BN, PO)), _rows((BN, 1))],
        out_shape=[jax.ShapeDtypeStruct((N, D_EMB), jnp.float32),
                   jax.ShapeDtypeStruct((N, PO), jnp.float32),
                   jax.ShapeDtypeStruct((N, 1), jnp.float32)],
    )(h, w_out, b_out, pw1, pb1, pw2, pb2, rw, rb)


def kernel(x, edge_index, W_in, b_in, Wmsg, Wz, Uz, bz, Wr, Ur, br,
           Wh, Uh, bh, W_out, b_out, Pw1, Pb1, Pw2, Pb2, Rw, Rb):
    src = edge_index[0]
    dst = edge_index[1]
    pad = EPAD - E
    srcp = jnp.concatenate([src, jnp.zeros((pad,), jnp.int32)])
    dstp = jnp.concatenate([dst, jnp.full((pad,), DUMP, jnp.int32)])
    src2 = jnp.concatenate([srcp, srcp + N]).reshape(NC * NS * NCHUNK, CHUNK)
    dst2 = dstp.reshape(NS * NCHUNK, CHUNK)

    b_in2 = b_in.reshape(1, H)
    b_out2 = b_out.reshape(1, D_EMB)
    pb1 = Pb1.reshape(1, PH)
    pb2 = Pb2.reshape(1, PO)
    rb = Rb.reshape(1, 1)

    h, mw = _tc_input_proj(x, W_in, b_in2, Wmsg[0])
    h_res = None
    for rnd in range(NLAYERS * NSTEPS):
        layer = rnd // NSTEPS
        step = rnd % NSTEPS
        if step == 0:
            h_res = h
        msg2 = _sc_segment_sum(mw.reshape(NC * N, HH), src2, dst2)
        last = rnd == NLAYERS * NSTEPS - 1
        wm_next = None if last else Wmsg[(rnd + 1) // NSTEPS]
        res = h_res if step == NSTEPS - 1 else None
        h, mw = _tc_gru(h, msg2, Wz[layer], Uz[layer], bz[layer].reshape(1, H),
                        Wr[layer], Ur[layer], br[layer].reshape(1, H),
                        Wh[layer], Uh[layer], bh[layer].reshape(1, H),
                        wm_next, res)
    node_emb, proj_emb, rank = _tc_tail(h, W_out, b_out2, Pw1, pb1, Pw2, pb2,
                                        Rw, rb)
    return node_emb, proj_emb, rank.reshape(-1)


# speedup vs baseline: 3.5078x; 1.4174x over previous
import functools

import jax
import jax.numpy as jnp
from jax import lax
from jax.experimental import pallas as pl
from jax.experimental.pallas import tpu as pltpu
from jax.experimental.pallas import tpu_sc as plsc

_dot = jnp.dot

N = 10000
E = 160000
D_IN = 256
H = 256
HH = 128
D_EMB = 64
NLAYERS = 3
NSTEPS = 5
PH = 256
PO = 64

NC = 2
NS = 16
CHUNK = 128
NCHUNK = 80
EPT = NCHUNK * CHUNK
EPAD = EPT * NS
DUMP = N
OPT = 624

BN = 1000
G = N // BN


NBUF = 2
RI = 4
ZPT = 632
NPAD = NS * ZPT


def _sc_body(mw_hbm, idx_hbm, out_hbm, idxr, b0, b1, acc,
             g0, g1, i0, i1, i2, i3):
    c = lax.axis_index("c")
    s = lax.axis_index("s")
    base = (c * NS + s) * NCHUNK
    bufs = (b0, b1)
    gsems = (g0, g1)
    isems = (i0, i1, i2, i3)

    zero16 = jnp.zeros((16,), jnp.float32)

    def zrow(i, carry):
        for j in range(HH // 16):
            b0[i, pl.ds(j * 16, 16)] = zero16
        return carry

    lax.fori_loop(0, CHUNK, zrow, 0)
    for b in range(ZPT // CHUNK):
        pltpu.sync_copy(b0, acc.at[pl.ds(s * ZPT + b * CHUNK, CHUNK)])
    rem = ZPT % CHUNK
    pltpu.sync_copy(b0.at[pl.ds(0, rem)],
                    acc.at[pl.ds(s * ZPT + (ZPT // CHUNK) * CHUNK, rem)])
    plsc.subcore_barrier()

    for r in range(RI):
        pltpu.async_copy(idx_hbm.at[base + r], idxr.at[r], isems[r])
    for b in range(NBUF):
        pltpu.make_async_copy(idx_hbm.at[base + b], idxr.at[b], isems[b]).wait()
        pltpu.async_copy(mw_hbm.at[idxr.at[b, 0]], bufs[b], gsems[b])

    def stage(i, bslot, islot):
        pltpu.make_async_copy(mw_hbm.at[idxr.at[islot, 0]], bufs[bslot],
                              gsems[bslot]).wait()
        pltpu.sync_copy(bufs[bslot], acc.at[idxr.at[islot, 1]], add=True)

    def group(g, carry):
        for b in range(RI):
            i = g * RI + b
            stage(i, b % NBUF, b)
            pltpu.async_copy(idx_hbm.at[base + i + RI], idxr.at[b], isems[b])
            nslot = (b + NBUF) % RI
            pltpu.make_async_copy(idx_hbm.at[base + i], idxr.at[nslot],
                                  isems[nslot]).wait()
            pltpu.async_copy(mw_hbm.at[idxr.at[nslot, 0]], bufs[b % NBUF],
                             gsems[b % NBUF])
        return carry

    lax.fori_loop(0, (NCHUNK - RI * 2) // RI, group, 0)
    for i in range(NCHUNK - RI * 2, NCHUNK):
        stage(i, i % NBUF, i % RI)
        if i + RI < NCHUNK:
            pltpu.async_copy(idx_hbm.at[base + i + RI], idxr.at[i % RI],
                             isems[i % RI])
        if i + NBUF < NCHUNK:
            nslot = (i + NBUF) % RI
            pltpu.make_async_copy(idx_hbm.at[base + i], idxr.at[nslot],
                                  isems[nslot]).wait()
            pltpu.async_copy(mw_hbm.at[idxr.at[nslot, 0]], bufs[i % NBUF],
                             gsems[i % NBUF])
    plsc.subcore_barrier()

    @pl.when(s < NS - 1)
    def _():
        pltpu.sync_copy(acc.at[pl.ds(s * OPT, OPT)],
                        out_hbm.at[pl.ds(c * N + s * OPT, OPT)])

    @pl.when(s == NS - 1)
    def _():
        pltpu.sync_copy(acc.at[pl.ds((NS - 1) * OPT, N - (NS - 1) * OPT)],
                        out_hbm.at[pl.ds(c * N + (NS - 1) * OPT,
                                         N - (NS - 1) * OPT)])


@functools.lru_cache(maxsize=1)
def _sc_segment_sum_kernel():
    mesh = plsc.VectorSubcoreMesh(
        core_axis_name="c", subcore_axis_name="s",
        num_cores=NC, num_subcores=NS)
    return pl.kernel(
        _sc_body,
        out_type=jax.ShapeDtypeStruct((NC * N, HH), jnp.float32),
        mesh=mesh,
        scratch_types=[
            pltpu.VMEM((RI, 2, CHUNK), jnp.int32),
            pltpu.VMEM((CHUNK, HH), jnp.float32),
            pltpu.VMEM((CHUNK, HH), jnp.float32),
            pltpu.VMEM_SHARED((NPAD, HH), jnp.float32),
            pltpu.SemaphoreType.DMA,
            pltpu.SemaphoreType.DMA,
            pltpu.SemaphoreType.DMA,
            pltpu.SemaphoreType.DMA,
            pltpu.SemaphoreType.DMA,
            pltpu.SemaphoreType.DMA,
        ],
    )


def _sc_segment_sum(mw2, idx2):
    return _sc_segment_sum_kernel()(mw2, idx2)


def _full(shape):
    return pl.BlockSpec(shape, lambda i: tuple(0 for _ in shape))


def _rows(shape):
    return pl.BlockSpec(shape, lambda i: (i,) + tuple(0 for _ in shape[1:]))


def _tc_input_proj(x, w_in, b_in, wm0):
    def body(x_ref, w_ref, b_ref, wm_ref, h_ref, mw_ref):
        h = jnp.maximum(_dot(x_ref[...], w_ref[...]) + b_ref[...], 0.0)
        h_ref[...] = h
        mw_ref[0] = _dot(h, wm_ref[:, :HH])
        mw_ref[1] = _dot(h, wm_ref[:, HH:])

    return pl.pallas_call(
        body,
        grid=(G,),
        in_specs=[_rows((BN, D_IN)), _full((D_IN, H)), _full((1, H)),
                  _full((H, H))],
        out_specs=[_rows((BN, H)),
                   pl.BlockSpec((NC, BN, HH), lambda i: (0, i, 0))],
        out_shape=[jax.ShapeDtypeStruct((N, H), jnp.float32),
                   jax.ShapeDtypeStruct((NC, N, HH), jnp.float32)],
    )(x, w_in, b_in, wm0)


def _tc_gru(h, msg2, wz, uz, bz, wr, ur, br, wh, uh, bh, wm_next, h_res):
    with_mw = wm_next is not None
    with_res = h_res is not None

    def body(*refs):
        it = iter(refs)
        h_ref = next(it); m0_ref = next(it); m1_ref = next(it)
        wz_ref = next(it); uz_ref = next(it); bz_ref = next(it)
        wr_ref = next(it); ur_ref = next(it); br_ref = next(it)
        wh_ref = next(it); uh_ref = next(it); bh_ref = next(it)
        wm_ref = next(it) if with_mw else None
        hr_ref = next(it) if with_res else None
        out_h = next(it)
        out_mw = next(it) if with_mw else None

        h = h_ref[...]
        m0 = m0_ref[...]
        m1 = m1_ref[...]
        z = jax.nn.sigmoid(_dot(m0, wz_ref[:HH, :]) + _dot(m1, wz_ref[HH:, :])
                           + _dot(h, uz_ref[...]) + bz_ref[...])
        r = jax.nn.sigmoid(_dot(m0, wr_ref[:HH, :]) + _dot(m1, wr_ref[HH:, :])
                           + _dot(h, ur_ref[...]) + br_ref[...])
        ht = jnp.tanh(_dot(m0, wh_ref[:HH, :]) + _dot(m1, wh_ref[HH:, :])
                      + _dot(r * h, uh_ref[...]) + bh_ref[...])
        hn = (1.0 - z) * h + z * ht
        if with_res:
            hn = hn + hr_ref[...]
        out_h[...] = hn
        if with_mw:
            out_mw[0] = _dot(hn, wm_ref[:, :HH])
            out_mw[1] = _dot(hn, wm_ref[:, HH:])

    in_specs = [_rows((BN, H)),
                pl.BlockSpec((BN, HH), lambda i: (i, 0)),
                pl.BlockSpec((BN, HH), lambda i: (G + i, 0)),
                _full((H, H)), _full((H, H)), _full((1, H)),
                _full((H, H)), _full((H, H)), _full((1, H)),
                _full((H, H)), _full((H, H)), _full((1, H))]
    args = [h, msg2, msg2, wz, uz, bz, wr, ur, br, wh, uh, bh]
    if with_mw:
        in_specs.append(_full((H, H)))
        args.append(wm_next)
    if with_res:
        in_specs.append(_rows((BN, H)))
        args.append(h_res)
    out_specs = [_rows((BN, H))]
    out_shape = [jax.ShapeDtypeStruct((N, H), jnp.float32)]
    if with_mw:
        out_specs.append(pl.BlockSpec((NC, BN, HH), lambda i: (0, i, 0)))
        out_shape.append(jax.ShapeDtypeStruct((NC, N, HH), jnp.float32))
    res = pl.pallas_call(
        body, grid=(G,), in_specs=in_specs, out_specs=out_specs,
        out_shape=out_shape,
    )(*args)
    return (res[0], res[1]) if with_mw else (res[0], None)


def _tc_tail(h, w_out, b_out, pw1, pb1, pw2, pb2, rw, rb):
    def body(h_ref, wo_ref, bo_ref, p1_ref, pb1_ref, p2_ref, pb2_ref,
             rw_ref, rb_ref, ne_ref, pe_ref, rk_ref):
        ne = _dot(h_ref[...], wo_ref[...]) + bo_ref[...]
        ne_ref[...] = ne
        hid = jnp.maximum(_dot(ne, p1_ref[...]) + pb1_ref[...], 0.0)
        pe_ref[...] = _dot(hid, p2_ref[...]) + pb2_ref[...]
        rk_ref[...] = _dot(ne, rw_ref[...]) + rb_ref[...]

    return pl.pallas_call(
        body,
        grid=(G,),
        in_specs=[_rows((BN, H)), _full((H, D_EMB)), _full((1, D_EMB)),
                  _full((D_EMB, PH)), _full((1, PH)),
                  _full((PH, PO)), _full((1, PO)),
                  _full((D_EMB, 1)), _full((1, 1))],
        out_specs=[_rows((BN, D_EMB)), _rows((BN, PO)), _rows((BN, 1))],
        out_shape=[jax.ShapeDtypeStruct((N, D_EMB), jnp.float32),
                   jax.ShapeDtypeStruct((N, PO), jnp.float32),
                   jax.ShapeDtypeStruct((N, 1), jnp.float32)],
    )(h, w_out, b_out, pw1, pb1, pw2, pb2, rw, rb)


def kernel(x, edge_index, W_in, b_in, Wmsg, Wz, Uz, bz, Wr, Ur, br,
           Wh, Uh, bh, W_out, b_out, Pw1, Pb1, Pw2, Pb2, Rw, Rb):
    src = edge_index[0]
    dst = edge_index[1]
    pad = EPAD - E
    srcp = jnp.concatenate([src, jnp.zeros((pad,), jnp.int32)])
    dstp = jnp.concatenate([dst, jnp.full((pad,), DUMP, jnp.int32)])
    src_t = srcp.reshape(NS, NCHUNK, CHUNK)
    dst_t = dstp.reshape(NS, NCHUNK, CHUNK)
    idx2 = jnp.stack(
        [jnp.stack([src_t + c * N, dst_t], axis=2) for c in range(NC)],
        axis=0).reshape(NC * NS * NCHUNK, 2, CHUNK)

    b_in2 = b_in.reshape(1, H)
    b_out2 = b_out.reshape(1, D_EMB)
    pb1 = Pb1.reshape(1, PH)
    pb2 = Pb2.reshape(1, PO)
    rb = Rb.reshape(1, 1)

    h, mw = _tc_input_proj(x, W_in, b_in2, Wmsg[0])
    h_res = None
    for rnd in range(NLAYERS * NSTEPS):
        layer = rnd // NSTEPS
        step = rnd % NSTEPS
        if step == 0:
            h_res = h
        msg2 = _sc_segment_sum(mw.reshape(NC * N, HH), idx2)
        last = rnd == NLAYERS * NSTEPS - 1
        wm_next = None if last else Wmsg[(rnd + 1) // NSTEPS]
        res = h_res if step == NSTEPS - 1 else None
        h, mw = _tc_gru(h, msg2, Wz[layer], Uz[layer], bz[layer].reshape(1, H),
                        Wr[layer], Ur[layer], br[layer].reshape(1, H),
                        Wh[layer], Uh[layer], bh[layer].reshape(1, H),
                        wm_next, res)
    node_emb, proj_emb, rank = _tc_tail(h, W_out, b_out2, Pw1, pb1, Pw2, pb2,
                                        Rw, rb)
    return node_emb, proj_emb, rank.reshape(-1)
